# Initial kernel scaffold; baseline (speedup 1.0000x reference)
import functools

import jax
import jax.numpy as jnp
from jax import lax
from jax.experimental import pallas as pl
from jax.experimental.pallas import tpu as pltpu
from jax.experimental.pallas import tpu_sc as plsc

N_NODES = 10000
N_EDGES = 320000
D_FEAT = 128
DH = D_FEAT // 2
NS = 16
EPT = N_EDGES // NS
K = 80
NCHUNK = EPT // K
ROWS_PT = N_NODES // NS
ZROWS = 125


def _sc_body(ego2, rowi, coli, vals, out,
             colbuf, rowbuf, idxbuf, valbuf, gbuf, zbuf, acc, gsem):
    core = lax.axis_index("c")
    tid = lax.axis_index("s")

    def zrow(r, carry):
        for j in range(DH // 16):
            zbuf[r, pl.ds(j * 16, 16)] = jnp.zeros((16,), jnp.float32)
        return carry

    lax.fori_loop(0, ZROWS, zrow, 0)
    for rep in range(ROWS_PT // ZROWS):
        pltpu.sync_copy(zbuf, acc.at[pl.ds(tid * ROWS_PT + rep * ZROWS, ZROWS)])
    plsc.subcore_barrier()

    def chunk(i, carry):
        base = tid * EPT + i * K
        pltpu.sync_copy(coli.at[pl.ds(base, K)], colbuf)
        pltpu.sync_copy(rowi.at[pl.ds(base, K)], rowbuf)
        pltpu.sync_copy(vals.at[pl.ds(base, K)], valbuf)
        for j in range(K // 16):
            idxbuf[pl.ds(j * 16, 16)] = colbuf[pl.ds(j * 16, 16)] * 2 + core
        pltpu.async_copy(ego2.at[idxbuf], gbuf, gsem).wait()

        def scale(e, c2):
            v = valbuf[e]
            for j in range(DH // 16):
                gbuf[e, pl.ds(j * 16, 16)] = gbuf[e, pl.ds(j * 16, 16)] * v
            return c2

        lax.fori_loop(0, K, scale, 0)
        pltpu.sync_copy(gbuf, acc.at[rowbuf], add=True)
        return carry

    lax.fori_loop(0, NCHUNK, chunk, 0)

    plsc.subcore_barrier()
    pltpu.sync_copy(acc.at[pl.ds(tid * ROWS_PT, ROWS_PT)],
                    out.at[core, pl.ds(tid * ROWS_PT, ROWS_PT)])


@jax.jit
def kernel(ego_embeddings, edge_index, adj_values):
    ego2 = ego_embeddings.reshape(2 * N_NODES, DH)
    rowi = edge_index[0]
    coli = edge_index[1]

    mesh = plsc.VectorSubcoreMesh(core_axis_name="c", subcore_axis_name="s")
    out = pl.kernel(
        _sc_body,
        out_type=jax.ShapeDtypeStruct((2, N_NODES, DH), jnp.float32),
        mesh=mesh,
        scratch_types=[
            pltpu.VMEM((K,), jnp.int32),
            pltpu.VMEM((K,), jnp.int32),
            pltpu.VMEM((K,), jnp.int32),
            pltpu.VMEM((K,), jnp.float32),
            pltpu.VMEM((K, DH), jnp.float32),
            pltpu.VMEM((ZROWS, DH), jnp.float32),
            pltpu.VMEM_SHARED((N_NODES, DH), jnp.float32),
            pltpu.SemaphoreType.DMA,
        ],
    )(ego2, rowi, coli, adj_values)

    return out.transpose(1, 0, 2).reshape(N_NODES, D_FEAT)


# speedup vs baseline: 4.5481x; 4.5481x over previous
import functools

import jax
import jax.numpy as jnp
from jax import lax
from jax.experimental import pallas as pl
from jax.experimental.pallas import tpu as pltpu
from jax.experimental.pallas import tpu_sc as plsc

N_NODES = 10000
N_EDGES = 320000
D_FEAT = 128
NS = 16
NC = 2
NW = NS * NC
EPT = N_EDGES // NW
K = 80
NCHUNK = EPT // K
NW_OUT = 10
ROWS_PT = N_NODES // NW_OUT
ZROWS = 125


def _sc_body(ego, rowi, coli, vals, out,
             colbuf, rowbuf, valbuf, gbuf, zbuf, acc, gsem):
    core = lax.axis_index("c")
    tid = lax.axis_index("s")
    wid = core * NS + tid

    def zrow(r, carry):
        for j in range(D_FEAT // 16):
            zbuf[r, pl.ds(j * 16, 16)] = jnp.zeros((16,), jnp.float32)
        return carry

    lax.fori_loop(0, ZROWS, zrow, 0)

    @pl.when(tid < NW_OUT)
    def _zero():
        for rep in range(ROWS_PT // ZROWS):
            pltpu.sync_copy(
                zbuf, acc.at[pl.ds(tid * ROWS_PT + rep * ZROWS, ZROWS)])

    plsc.subcore_barrier()

    def chunk(i, carry):
        base = wid * EPT + i * K
        pltpu.sync_copy(coli.at[pl.ds(base, K)], colbuf)
        pltpu.sync_copy(rowi.at[pl.ds(base, K)], rowbuf)
        pltpu.sync_copy(vals.at[pl.ds(base, K)], valbuf)
        pltpu.async_copy(ego.at[colbuf], gbuf, gsem).wait()

        def scale(g, c2):
            v16 = valbuf[pl.ds(g * 16, 16)]
            for i2 in range(16):
                v = v16[i2]
                e = g * 16 + i2
                for j in range(D_FEAT // 16):
                    gbuf[e, pl.ds(j * 16, 16)] = (
                        gbuf[e, pl.ds(j * 16, 16)] * v)
            return c2

        lax.fori_loop(0, K // 16, scale, 0)
        pltpu.sync_copy(gbuf, acc.at[rowbuf], add=True)
        return carry

    lax.fori_loop(0, NCHUNK, chunk, 0)

    plsc.subcore_barrier()

    @pl.when(tid < NW_OUT)
    def _writeout():
        pltpu.sync_copy(acc.at[pl.ds(tid * ROWS_PT, ROWS_PT)],
                        out.at[core, pl.ds(tid * ROWS_PT, ROWS_PT)])


def _combine_body(p_ref, o_ref):
    o_ref[...] = p_ref[0] + p_ref[1]


@jax.jit
def kernel(ego_embeddings, edge_index, adj_values):
    rowi = edge_index[0]
    coli = edge_index[1]

    mesh = plsc.VectorSubcoreMesh(core_axis_name="c", subcore_axis_name="s")
    partials = pl.kernel(
        _sc_body,
        out_type=jax.ShapeDtypeStruct((NC, N_NODES, D_FEAT), jnp.float32),
        mesh=mesh,
        scratch_types=[
            pltpu.VMEM((K,), jnp.int32),
            pltpu.VMEM((K,), jnp.int32),
            pltpu.VMEM((K,), jnp.float32),
            pltpu.VMEM((K, D_FEAT), jnp.float32),
            pltpu.VMEM((ZROWS, D_FEAT), jnp.float32),
            pltpu.VMEM_SHARED((N_NODES, D_FEAT), jnp.float32),
            pltpu.SemaphoreType.DMA,
        ],
    )(ego_embeddings, rowi, coli, adj_values)

    rows_blk = 2000
    return pl.pallas_call(
        _combine_body,
        grid=(N_NODES // rows_blk,),
        in_specs=[pl.BlockSpec((NC, rows_blk, D_FEAT), lambda i: (0, i, 0))],
        out_specs=pl.BlockSpec((rows_blk, D_FEAT), lambda i: (i, 0)),
        out_shape=jax.ShapeDtypeStruct((N_NODES, D_FEAT), jnp.float32),
    )(partials)
